# Initial kernel scaffold; baseline (speedup 1.0000x reference)
import jax
import jax.numpy as jnp
from jax.experimental import pallas as pl


def kernel(x, edge_index, edge_attr, batch, Wl0, Wr0, We0, att0, b0, Wl1, Wr1, We1, att1, b1, Wl2, Wr2, We2, att2, b2, llW, llb, flW, flb):
    raise NotImplementedError("write your pallas kernel here")


# speedup vs baseline: 2.4427x; 2.4427x over previous
import functools

import jax
import jax.numpy as jnp
from jax import lax
from jax.experimental import pallas as pl
from jax.experimental.pallas import tpu as pltpu
from jax.experimental.pallas import tpu_sc as plsc

N = 10000
E = 160000
C = 256
NC, NS, L = 2, 16, 16
NW = NC * NS
NPT = 320
NPAD = NW * NPT
CH = 32
DCH = 2000
FLB = 2048
EP = E + NW * CH

_mesh = plsc.VectorSubcoreMesh(
    core_axis_name="c", subcore_axis_name="s", num_cores=NC, num_subcores=NS)
_sc_params = pltpu.CompilerParams(needs_layout_passes=False)

_f32 = jnp.float32
_i32 = jnp.int32


def _wid():
    return lax.axis_index("s") * NC + lax.axis_index("c")


def _lane0():
    return lax.iota(_i32, L) == 0


def _sstore(ref, pos, val):
    idx = jnp.broadcast_to(pos, (L,))
    v = jnp.broadcast_to(val, (L,))
    plsc.store_scatter(ref, [idx], v, mask=_lane0())


def _gat(ref, pos):
    return plsc.load_gather(ref, [jnp.broadcast_to(pos, (L,))])


def _extract(ref, pos):
    return jnp.max(_gat(ref, pos))


@functools.partial(
    pl.kernel,
    out_type=jax.ShapeDtypeStruct((NW * 8,), _i32),
    mesh=_mesh,
    compiler_params=_sc_params,
    scratch_types=[pltpu.VMEM((DCH,), _i32), pltpu.VMEM((16,), _i32)],
)
def _g1(dst_h, cnt8_h, dbuf, cbuf):
    wid = _wid()
    lo = wid * NPT
    hi = lo + NPT

    def outer(o, cnt16):
        pltpu.sync_copy(dst_h.at[pl.ds(o * DCH, DCH)], dbuf)

        def inner(j, c16):
            d = dbuf[pl.ds(j * L, L)]
            msk = (d >= lo) & (d < hi)
            return c16 + jnp.where(msk, 1.0, 0.0).astype(_f32)

        return lax.fori_loop(0, DCH // L, inner, cnt16)

    cnt16 = lax.fori_loop(0, E // DCH, outer, jnp.zeros((L,), _f32))
    total = jnp.sum(cnt16).astype(_i32)
    cbuf[...] = jnp.where(_lane0(), jnp.broadcast_to(total, (L,)), 0)
    pltpu.sync_copy(cbuf.at[pl.ds(0, 8)], cnt8_h.at[pl.ds(pl.multiple_of(wid * 8, 8), 8)])


@functools.partial(
    pl.kernel,
    out_type=[
        jax.ShapeDtypeStruct((EP,), _i32),
        jax.ShapeDtypeStruct((EP,), _i32),
        jax.ShapeDtypeStruct((EP,), _i32),
        jax.ShapeDtypeStruct((NW * 8,), _i32),
    ],
    mesh=_mesh,
    compiler_params=_sc_params,
    scratch_types=[
        pltpu.VMEM((DCH,), _i32),
        pltpu.VMEM((DCH,), _i32),
        pltpu.VMEM((FLB + CH,), _i32),
        pltpu.VMEM((FLB + CH,), _i32),
        pltpu.VMEM((FLB + CH,), _i32),
        pltpu.VMEM((NW * 8,), _i32),
        pltpu.VMEM((16,), _i32),
    ],
)
def _g2(dst_h, src_h, cnt8_h, perm_h, srcp_h, dstp_h, offs8_h,
        dbuf, sbuf, idb, srb, dsb, cntb, obuf):
    wid = _wid()
    lo = wid * NPT
    hi = lo + NPT
    pltpu.sync_copy(cnt8_h, cntb)

    def acc_off(u, acc):
        cu = _extract(cntb, u * 8)
        return acc + ((cu + CH - 1) // CH) * CH

    my_off = lax.fori_loop(0, wid, acc_off, jnp.int32(0))
    obuf[...] = jnp.where(_lane0(), jnp.broadcast_to(my_off, (L,)), 0)
    pltpu.sync_copy(obuf.at[pl.ds(0, 8)], offs8_h.at[pl.ds(pl.multiple_of(wid * 8, 8), 8)])

    lanes = lax.iota(_i32, L)

    def outer(o, carry):
        pltpu.sync_copy(dst_h.at[pl.ds(o * DCH, DCH)], dbuf)
        pltpu.sync_copy(src_h.at[pl.ds(o * DCH, DCH)], sbuf)

        def inner(j, carry):
            fill, wpos = carry
            sl = pl.ds(j * L, L)
            d = dbuf[sl]
            s = sbuf[sl]
            msk = (d >= lo) & (d < hi)
            ids = jnp.broadcast_to(o * DCH + j * L, (L,)) + lanes
            plsc.store_compressed(idb.at[pl.ds(fill, L)], ids, mask=msk)
            plsc.store_compressed(srb.at[pl.ds(fill, L)], s, mask=msk)
            plsc.store_compressed(dsb.at[pl.ds(fill, L)], d, mask=msk)
            fill = fill + jnp.sum(jnp.where(msk, 1.0, 0.0)).astype(_i32)
            flush = fill >= FLB

            @pl.when(flush)
            def _():
                pltpu.sync_copy(idb.at[pl.ds(0, FLB)],
                                perm_h.at[pl.ds(pl.multiple_of(wpos, 8), FLB)])
                pltpu.sync_copy(srb.at[pl.ds(0, FLB)],
                                srcp_h.at[pl.ds(pl.multiple_of(wpos, 8), FLB)])
                pltpu.sync_copy(dsb.at[pl.ds(0, FLB)],
                                dstp_h.at[pl.ds(pl.multiple_of(wpos, 8), FLB)])
                idb[pl.ds(0, L)] = idb[pl.ds(FLB, L)]
                srb[pl.ds(0, L)] = srb[pl.ds(FLB, L)]
                dsb[pl.ds(0, L)] = dsb[pl.ds(FLB, L)]

            fill = jnp.where(flush, fill - FLB, fill)
            wpos = jnp.where(flush, wpos + FLB, wpos)
            return fill, wpos

        return lax.fori_loop(0, DCH // L, inner, carry)

    fill, wpos = lax.fori_loop(0, E // DCH, outer,
                               (jnp.int32(0), my_off))

    zi = jnp.zeros((L,), _i32)
    lov = jnp.broadcast_to(lo, (L,))
    for t in (0, L):
        idb[pl.ds(fill + t, L)] = zi
        srb[pl.ds(fill + t, L)] = zi
        dsb[pl.ds(fill + t, L)] = lov
    nfl = (fill + CH - 1) // CH

    def fl(k, _):
        pltpu.sync_copy(idb.at[pl.ds(k * CH, CH)],
                        perm_h.at[pl.ds(pl.multiple_of(wpos + k * CH, 8), CH)])
        pltpu.sync_copy(srb.at[pl.ds(k * CH, CH)],
                        srcp_h.at[pl.ds(pl.multiple_of(wpos + k * CH, 8), CH)])
        pltpu.sync_copy(dsb.at[pl.ds(k * CH, CH)],
                        dstp_h.at[pl.ds(pl.multiple_of(wpos + k * CH, 8), CH)])
        return 0

    lax.fori_loop(0, nfl, fl, 0)


def _make_stage_b(H):
    @functools.partial(
        pl.kernel,
        out_type=[
            jax.ShapeDtypeStruct((H * NPAD * C,), _f32),
            jax.ShapeDtypeStruct((EP,), _f32),
        ],
        mesh=_mesh,
        compiler_params=_sc_params,
        scratch_types=[
            pltpu.VMEM((NPT * C,), _f32),
            pltpu.VMEM((CH, C), _f32),
            pltpu.VMEM((CH, C), _f32),
            pltpu.VMEM((CH, C), _f32),
            pltpu.VMEM((NPT,), _f32),
            pltpu.VMEM((NPT,), _f32),
            pltpu.VMEM((CH,), _i32),
            pltpu.VMEM((CH,), _i32),
            pltpu.VMEM((CH,), _i32),
            pltpu.VMEM((CH,), _i32),
            pltpu.VMEM((CH,), _f32),
            pltpu.VMEM((CH,), _f32),
            pltpu.VMEM((C,), _f32),
            pltpu.VMEM((C,), _f32),
            pltpu.VMEM((NW * 8,), _i32),
            pltpu.VMEM((NW * 8,), _i32),
            pltpu.SemaphoreType.DMA,
        ],
    )
    def stage_b(xl2, xr2, ea2, attf, bf, srcp, dstp, perm, cnt8, offs8,
                out1, ascr, m, xlr, xrr, ear, amax, den,
                srcv, dstv, pmv, idxb, abuf, exb, attv, bv, cntb, offb, sem):
        wid = _wid()
        base = wid * NPT
        pltpu.sync_copy(cnt8, cntb)
        pltpu.sync_copy(offs8, offb)
        my_cnt = _extract(cntb, wid * 8)
        my_off = _extract(offb, wid * 8)
        nch = (my_cnt + CH - 1) // CH
        zf = jnp.zeros((L,), _f32)

        for h in range(H):
            pltpu.sync_copy(attf.at[pl.ds(h * C, C)], attv)
            pltpu.sync_copy(bf.at[pl.ds(h * C, C)], bv)

            def zi(i, _):
                amax[pl.ds(i * L, L)] = jnp.full((L,), -1e30, _f32)
                den[pl.ds(i * L, L)] = zf
                return 0

            lax.fori_loop(0, NPT // L, zi, 0)

            def zm(i, _):
                for q in range(16):
                    m[pl.ds(i * 256 + q * L, L)] = zf
                return 0

            lax.fori_loop(0, NPT * C // 256, zm, 0)

            def p1(i, _):
                g = my_off + i * CH
                pltpu.sync_copy(srcp.at[pl.ds(pl.multiple_of(g, 8), CH)], srcv)
                pltpu.sync_copy(dstp.at[pl.ds(pl.multiple_of(g, 8), CH)], dstv)
                pltpu.sync_copy(perm.at[pl.ds(pl.multiple_of(g, 8), CH)], pmv)
                for q in range(CH // L):
                    sl = pl.ds(q * L, L)
                    idxb[sl] = srcv[sl] + h * NPAD
                pltpu.async_copy(xl2.at[idxb], xlr, sem).wait()
                for q in range(CH // L):
                    sl = pl.ds(q * L, L)
                    idxb[sl] = dstv[sl] + h * NPAD
                pltpu.async_copy(xr2.at[idxb], xrr, sem).wait()
                for q in range(CH // L):
                    sl = pl.ds(q * L, L)
                    idxb[sl] = pmv[sl] + h * E
                pltpu.async_copy(ea2.at[idxb], ear, sem).wait()

                def pe(e, _):
                    acc = zf
                    for j in range(C // L):
                        sl = pl.ds(j * L, L)
                        s = xlr[e, sl] + xrr[e, sl] + ear[e, sl]
                        s = jnp.where(s > 0, s, s * 0.2)
                        acc = acc + s * attv[sl]
                    al = jnp.sum(acc)
                    _sstore(abuf, e, al)

                    @pl.when(i * CH + e < my_cnt)
                    def _():
                        dlv = _gat(dstv, e) - base
                        cur = plsc.load_gather(amax, [dlv])
                        new = jnp.maximum(cur, jnp.broadcast_to(al, (L,)))
                        plsc.store_scatter(amax, [dlv], new)

                    return 0

                lax.fori_loop(0, CH, pe, 0)
                pltpu.sync_copy(abuf, ascr.at[pl.ds(pl.multiple_of(g, 8), CH)])
                return 0

            lax.fori_loop(0, nch, p1, 0)

            def p2(i, _):
                g = my_off + i * CH
                pltpu.sync_copy(srcp.at[pl.ds(pl.multiple_of(g, 8), CH)], srcv)
                pltpu.sync_copy(dstp.at[pl.ds(pl.multiple_of(g, 8), CH)], dstv)
                pltpu.sync_copy(ascr.at[pl.ds(pl.multiple_of(g, 8), CH)], abuf)
                for q in range(CH // L):
                    sl = pl.ds(q * L, L)
                    idxb[sl] = srcv[sl] + h * NPAD
                pltpu.async_copy(xl2.at[idxb], xlr, sem).wait()
                for q in range(CH // L):
                    sl = pl.ds(q * L, L)
                    dlv = dstv[sl] - base
                    mx = plsc.load_gather(amax, [dlv])
                    exb[sl] = jnp.exp(abuf[sl] - mx)

                def pe2(e, _):
                    @pl.when(i * CH + e < my_cnt)
                    def _():
                        exv = _gat(exb, e)
                        dlv = _gat(dstv, e) - base
                        cur = plsc.load_gather(den, [dlv])
                        plsc.store_scatter(den, [dlv], cur + exv)
                        mo = jnp.max(dlv) * C
                        for j in range(C // L):
                            sl2 = pl.ds(mo + j * L, L)
                            m[sl2] = m[sl2] + exv * xlr[e, pl.ds(j * L, L)]

                    return 0

                lax.fori_loop(0, CH, pe2, 0)
                return 0

            lax.fori_loop(0, nch, p2, 0)

            def p3(n, _):
                rv = 1.0 / (_gat(den, n) + 1e-16)
                mo = n * C
                for j in range(C // L):
                    sl = pl.ds(mo + j * L, L)
                    m[sl] = m[sl] * rv + bv[pl.ds(j * L, L)]
                return 0

            lax.fori_loop(0, NPT, p3, 0)
            pltpu.sync_copy(
                m, out1.at[pl.ds(pl.multiple_of((h * NPAD + base) * C, 8), NPT * C)])

    return stage_b


_stage_b4 = _make_stage_b(4)
_stage_b1 = _make_stage_b(1)


def _mm(a3, w4):
    KH, M, Cin = a3.shape
    _, HO, _, CO = w4.shape
    BM = 640

    def body(a_ref, w_ref, o_ref):
        kh = pl.program_id(2)
        part = jnp.dot(a_ref[0], w_ref[0, 0],
                       preferred_element_type=_f32)

        @pl.when(kh == 0)
        def _():
            o_ref[0] = part

        @pl.when(kh != 0)
        def _():
            o_ref[0] = o_ref[0] + part

    return pl.pallas_call(
        body,
        grid=(HO, M // BM, KH),
        in_specs=[
            pl.BlockSpec((1, BM, Cin), lambda ho, i, kh: (kh, i, 0)),
            pl.BlockSpec((1, 1, Cin, CO), lambda ho, i, kh: (kh, ho, 0, 0)),
        ],
        out_specs=pl.BlockSpec((1, BM, CO), lambda ho, i, kh: (ho, i, 0)),
        out_shape=jax.ShapeDtypeStruct((HO, M, CO), _f32),
    )(a3, w4)


def _head(h2, llW, llb, flW, flb):
    M = h2.shape[0]
    BM = 640

    def body(h_ref, lw_ref, lb_ref, fw_ref, fb_ref, o_ref):
        t = jnp.dot(h_ref[...], lw_ref[...], preferred_element_type=_f32)
        t = t + lb_ref[...]
        u = jnp.dot(t, fw_ref[...], preferred_element_type=_f32)
        u = u + fb_ref[...]
        o_ref[...] = jax.nn.sigmoid(u)

    return pl.pallas_call(
        body,
        grid=(M // BM,),
        in_specs=[
            pl.BlockSpec((BM, 256), lambda i: (i, 0)),
            pl.BlockSpec((256, 128), lambda i: (0, 0)),
            pl.BlockSpec((1, 128), lambda i: (0, 0)),
            pl.BlockSpec((128, 1), lambda i: (0, 0)),
            pl.BlockSpec((1, 1), lambda i: (0, 0)),
        ],
        out_specs=pl.BlockSpec((BM, 1), lambda i: (i, 0)),
        out_shape=jax.ShapeDtypeStruct((M, 1), _f32),
    )(h2, llW, llb, flW, flb)


def kernel(x, edge_index, edge_attr, batch,
           Wl0, Wr0, We0, att0, b0,
           Wl1, Wr1, We1, att1, b1,
           Wl2, Wr2, We2, att2, b2,
           llW, llb, flW, flb):
    src = edge_index[0]
    dst = edge_index[1]

    cnt8 = _g1(dst)
    perm, srcp, dstp, offs8 = _g2(dst, src, cnt8)

    x_p = jnp.pad(x, ((0, NPAD - N), (0, 0)))
    h = x_p[None]
    ea_in = edge_attr[None]

    layer_params = [
        (Wl0, Wr0, We0, att0, b0, 4, _stage_b4),
        (Wl1, Wr1, We1, att1, b1, 4, _stage_b4),
        (Wl2, Wr2, We2, att2, b2, 1, _stage_b1),
    ]
    for Wl, Wr, We, att, b, H, sb in layer_params:
        KH = h.shape[0]
        wl4 = Wl.reshape(KH, C, H, C).transpose(0, 2, 1, 3)
        wr4 = Wr.reshape(KH, C, H, C).transpose(0, 2, 1, 3)
        we4 = We.reshape(1, 16, H, C).transpose(0, 2, 1, 3)
        xl3 = _mm(h, wl4)
        xr3 = _mm(h, wr4)
        ea3 = _mm(ea_in, we4)
        out1, _unused = sb(
            xl3.reshape(H * NPAD, C), xr3.reshape(H * NPAD, C),
            ea3.reshape(H * E, C), att.reshape(H * C),
            b.reshape(H * C), srcp, dstp, perm, cnt8, offs8)
        h = out1.reshape(H, NPAD, C)

    out = _head(h[0], llW, llb.reshape(1, 128), flW, flb.reshape(1, 1))
    return out[:N]


# speedup vs baseline: 3.0748x; 1.2588x over previous
import functools

import jax
import jax.numpy as jnp
from jax import lax
from jax.experimental import pallas as pl
from jax.experimental.pallas import tpu as pltpu
from jax.experimental.pallas import tpu_sc as plsc

N = 10000
E = 160000
C = 256
NC, NS, L = 2, 16, 16
NW = NC * NS
NPT = 320
NPAD = NW * NPT
CH = 32
DCH = 2000
FLB = 2048
EP = E + NW * CH

_mesh = plsc.VectorSubcoreMesh(
    core_axis_name="c", subcore_axis_name="s", num_cores=NC, num_subcores=NS)
_sc_params = pltpu.CompilerParams(needs_layout_passes=False)

_f32 = jnp.float32
_i32 = jnp.int32


def _wid():
    return lax.axis_index("s") * NC + lax.axis_index("c")


def _lane0():
    return lax.iota(_i32, L) == 0


def _sstore(ref, pos, val):
    idx = jnp.broadcast_to(pos, (L,))
    v = jnp.broadcast_to(val, (L,))
    plsc.store_scatter(ref, [idx], v, mask=_lane0())


def _gat(ref, pos):
    return plsc.load_gather(ref, [jnp.broadcast_to(pos, (L,))])


def _extract(ref, pos):
    return jnp.max(_gat(ref, pos))


@functools.partial(
    pl.kernel,
    out_type=jax.ShapeDtypeStruct((NW * 8,), _i32),
    mesh=_mesh,
    compiler_params=_sc_params,
    scratch_types=[pltpu.VMEM((DCH,), _i32), pltpu.VMEM((16,), _i32)],
)
def _g1(dst_h, cnt8_h, dbuf, cbuf):
    wid = _wid()
    lo = wid * NPT
    hi = lo + NPT

    def outer(o, cnt16):
        pltpu.sync_copy(dst_h.at[pl.ds(o * DCH, DCH)], dbuf)

        def inner(j, c16):
            d = dbuf[pl.ds(j * L, L)]
            msk = (d >= lo) & (d < hi)
            return c16 + jnp.where(msk, 1.0, 0.0).astype(_f32)

        return lax.fori_loop(0, DCH // L, inner, cnt16)

    cnt16 = lax.fori_loop(0, E // DCH, outer, jnp.zeros((L,), _f32))
    total = jnp.sum(cnt16).astype(_i32)
    cbuf[...] = jnp.where(_lane0(), jnp.broadcast_to(total, (L,)), 0)
    pltpu.sync_copy(cbuf.at[pl.ds(0, 8)], cnt8_h.at[pl.ds(pl.multiple_of(wid * 8, 8), 8)])


@functools.partial(
    pl.kernel,
    out_type=[
        jax.ShapeDtypeStruct((EP,), _i32),
        jax.ShapeDtypeStruct((EP,), _i32),
        jax.ShapeDtypeStruct((EP,), _i32),
        jax.ShapeDtypeStruct((NW * 8,), _i32),
    ],
    mesh=_mesh,
    compiler_params=_sc_params,
    scratch_types=[
        pltpu.VMEM((DCH,), _i32),
        pltpu.VMEM((DCH,), _i32),
        pltpu.VMEM((FLB + CH,), _i32),
        pltpu.VMEM((FLB + CH,), _i32),
        pltpu.VMEM((FLB + CH,), _i32),
        pltpu.VMEM((NW * 8,), _i32),
        pltpu.VMEM((16,), _i32),
    ],
)
def _g2(dst_h, src_h, cnt8_h, perm_h, srcp_h, dstp_h, offs8_h,
        dbuf, sbuf, idb, srb, dsb, cntb, obuf):
    wid = _wid()
    lo = wid * NPT
    hi = lo + NPT
    pltpu.sync_copy(cnt8_h, cntb)

    def acc_off(u, acc):
        cu = _extract(cntb, u * 8)
        return acc + ((cu + CH - 1) // CH) * CH

    my_off = lax.fori_loop(0, wid, acc_off, jnp.int32(0))
    obuf[...] = jnp.where(_lane0(), jnp.broadcast_to(my_off, (L,)), 0)
    pltpu.sync_copy(obuf.at[pl.ds(0, 8)], offs8_h.at[pl.ds(pl.multiple_of(wid * 8, 8), 8)])

    lanes = lax.iota(_i32, L)

    def outer(o, carry):
        pltpu.sync_copy(dst_h.at[pl.ds(o * DCH, DCH)], dbuf)
        pltpu.sync_copy(src_h.at[pl.ds(o * DCH, DCH)], sbuf)

        def inner(j, carry):
            fill, wpos = carry
            sl = pl.ds(j * L, L)
            d = dbuf[sl]
            s = sbuf[sl]
            msk = (d >= lo) & (d < hi)
            ids = jnp.broadcast_to(o * DCH + j * L, (L,)) + lanes
            plsc.store_compressed(idb.at[pl.ds(fill, L)], ids, mask=msk)
            plsc.store_compressed(srb.at[pl.ds(fill, L)], s, mask=msk)
            plsc.store_compressed(dsb.at[pl.ds(fill, L)], d, mask=msk)
            fill = fill + jnp.sum(jnp.where(msk, 1.0, 0.0)).astype(_i32)
            flush = fill >= FLB

            @pl.when(flush)
            def _():
                pltpu.sync_copy(idb.at[pl.ds(0, FLB)],
                                perm_h.at[pl.ds(pl.multiple_of(wpos, 8), FLB)])
                pltpu.sync_copy(srb.at[pl.ds(0, FLB)],
                                srcp_h.at[pl.ds(pl.multiple_of(wpos, 8), FLB)])
                pltpu.sync_copy(dsb.at[pl.ds(0, FLB)],
                                dstp_h.at[pl.ds(pl.multiple_of(wpos, 8), FLB)])
                idb[pl.ds(0, L)] = idb[pl.ds(FLB, L)]
                srb[pl.ds(0, L)] = srb[pl.ds(FLB, L)]
                dsb[pl.ds(0, L)] = dsb[pl.ds(FLB, L)]

            fill = jnp.where(flush, fill - FLB, fill)
            wpos = jnp.where(flush, wpos + FLB, wpos)
            return fill, wpos

        return lax.fori_loop(0, DCH // L, inner, carry)

    fill, wpos = lax.fori_loop(0, E // DCH, outer,
                               (jnp.int32(0), my_off))

    zi = jnp.zeros((L,), _i32)
    lov = jnp.broadcast_to(lo, (L,))
    for t in (0, L):
        idb[pl.ds(fill + t, L)] = zi
        srb[pl.ds(fill + t, L)] = zi
        dsb[pl.ds(fill + t, L)] = lov
    nfl = (fill + CH - 1) // CH

    def fl(k, _):
        pltpu.sync_copy(idb.at[pl.ds(k * CH, CH)],
                        perm_h.at[pl.ds(pl.multiple_of(wpos + k * CH, 8), CH)])
        pltpu.sync_copy(srb.at[pl.ds(k * CH, CH)],
                        srcp_h.at[pl.ds(pl.multiple_of(wpos + k * CH, 8), CH)])
        pltpu.sync_copy(dsb.at[pl.ds(k * CH, CH)],
                        dstp_h.at[pl.ds(pl.multiple_of(wpos + k * CH, 8), CH)])
        return 0

    lax.fori_loop(0, nfl, fl, 0)


def _make_stage_b(H):
    @functools.partial(
        pl.kernel,
        out_type=[
            jax.ShapeDtypeStruct((H * NPAD * C,), _f32),
            jax.ShapeDtypeStruct((EP,), _f32),
        ],
        mesh=_mesh,
        compiler_params=_sc_params,
        scratch_types=[
            pltpu.VMEM((NPT * C,), _f32),
            pltpu.VMEM((CH, C), _f32),
            pltpu.VMEM((CH, C), _f32),
            pltpu.VMEM((CH, C), _f32),
            pltpu.VMEM((NPT,), _f32),
            pltpu.VMEM((NPT,), _f32),
            pltpu.VMEM((CH,), _i32),
            pltpu.VMEM((CH,), _i32),
            pltpu.VMEM((CH,), _i32),
            pltpu.VMEM((CH,), _i32),
            pltpu.VMEM((CH,), _i32),
            pltpu.VMEM((CH,), _i32),
            pltpu.VMEM((CH,), _f32),
            pltpu.VMEM((CH,), _f32),
            pltpu.VMEM((C,), _f32),
            pltpu.VMEM((C,), _f32),
            pltpu.VMEM((NW * 8,), _i32),
            pltpu.VMEM((NW * 8,), _i32),
            pltpu.SemaphoreType.DMA,
        ],
    )
    def stage_b(xl2, xr2, ea2, attf, bf, srcp, dstp, perm, cnt8, offs8,
                out1, ascr, m, xlr, xrr, ear, amax, den,
                srcv, dstv, pmv, idxb, idxb2, idxb3, abuf, exb, attv, bv,
                cntb, offb, sem):
        wid = _wid()
        base = wid * NPT
        pltpu.sync_copy(cnt8, cntb)
        pltpu.sync_copy(offs8, offb)
        my_cnt = _extract(cntb, wid * 8)
        my_off = _extract(offb, wid * 8)
        nch = (my_cnt + CH - 1) // CH
        zf = jnp.zeros((L,), _f32)

        for h in range(H):
            pltpu.sync_copy(attf.at[pl.ds(h * C, C)], attv)
            pltpu.sync_copy(bf.at[pl.ds(h * C, C)], bv)

            def zi(i, _):
                amax[pl.ds(i * L, L)] = jnp.full((L,), -1e30, _f32)
                den[pl.ds(i * L, L)] = zf
                return 0

            lax.fori_loop(0, NPT // L, zi, 0)

            def zm(i, _):
                for q in range(16):
                    m[pl.ds(i * 256 + q * L, L)] = zf
                return 0

            lax.fori_loop(0, NPT * C // 256, zm, 0)

            def p1(i, _):
                g = my_off + i * CH
                c1 = pltpu.async_copy(
                    srcp.at[pl.ds(pl.multiple_of(g, 8), CH)], srcv, sem)
                c2 = pltpu.async_copy(
                    dstp.at[pl.ds(pl.multiple_of(g, 8), CH)], dstv, sem)
                c3 = pltpu.async_copy(
                    perm.at[pl.ds(pl.multiple_of(g, 8), CH)], pmv, sem)
                c1.wait()
                c2.wait()
                c3.wait()
                for q in range(CH // L):
                    sl = pl.ds(q * L, L)
                    idxb[sl] = srcv[sl] + h * NPAD
                    idxb2[sl] = dstv[sl] + h * NPAD
                    idxb3[sl] = pmv[sl] + h * E
                g1 = pltpu.async_copy(xl2.at[idxb], xlr, sem)
                g2 = pltpu.async_copy(xr2.at[idxb2], xrr, sem)
                g3 = pltpu.async_copy(ea2.at[idxb3], ear, sem)
                g1.wait()
                g2.wait()
                g3.wait()

                def pe(e, _):
                    accs = [zf, zf, zf, zf]
                    for j in range(C // L):
                        sl = pl.ds(j * L, L)
                        s = xlr[e, sl] + xrr[e, sl] + ear[e, sl]
                        s = jnp.where(s > 0, s, s * 0.2)
                        accs[j % 4] = accs[j % 4] + s * attv[sl]
                    al = jnp.sum((accs[0] + accs[1]) + (accs[2] + accs[3]))
                    _sstore(abuf, e, al)

                    @pl.when(i * CH + e < my_cnt)
                    def _():
                        dlv = _gat(dstv, e) - base
                        cur = plsc.load_gather(amax, [dlv])
                        new = jnp.maximum(cur, jnp.broadcast_to(al, (L,)))
                        plsc.store_scatter(amax, [dlv], new)

                    return 0

                lax.fori_loop(0, CH, pe, 0)
                pltpu.sync_copy(abuf, ascr.at[pl.ds(pl.multiple_of(g, 8), CH)])
                return 0

            lax.fori_loop(0, nch, p1, 0)

            def p2(i, _):
                g = my_off + i * CH
                c1 = pltpu.async_copy(
                    srcp.at[pl.ds(pl.multiple_of(g, 8), CH)], srcv, sem)
                c2 = pltpu.async_copy(
                    dstp.at[pl.ds(pl.multiple_of(g, 8), CH)], dstv, sem)
                c3 = pltpu.async_copy(
                    ascr.at[pl.ds(pl.multiple_of(g, 8), CH)], abuf, sem)
                c1.wait()
                c2.wait()
                c3.wait()
                for q in range(CH // L):
                    sl = pl.ds(q * L, L)
                    idxb[sl] = srcv[sl] + h * NPAD
                pltpu.async_copy(xl2.at[idxb], xlr, sem).wait()
                for q in range(CH // L):
                    sl = pl.ds(q * L, L)
                    dlv = dstv[sl] - base
                    mx = plsc.load_gather(amax, [dlv])
                    exb[sl] = jnp.exp(abuf[sl] - mx)

                def pe2(e, _):
                    @pl.when(i * CH + e < my_cnt)
                    def _():
                        exv = _gat(exb, e)
                        dlv = _gat(dstv, e) - base
                        cur = plsc.load_gather(den, [dlv])
                        plsc.store_scatter(den, [dlv], cur + exv)
                        mo = jnp.max(dlv) * C
                        for j in range(C // L):
                            sl2 = pl.ds(mo + j * L, L)
                            m[sl2] = m[sl2] + exv * xlr[e, pl.ds(j * L, L)]

                    return 0

                lax.fori_loop(0, CH, pe2, 0)
                return 0

            lax.fori_loop(0, nch, p2, 0)

            def p3(n, _):
                rv = 1.0 / (_gat(den, n) + 1e-16)
                mo = n * C
                for j in range(C // L):
                    sl = pl.ds(mo + j * L, L)
                    m[sl] = m[sl] * rv + bv[pl.ds(j * L, L)]
                return 0

            lax.fori_loop(0, NPT, p3, 0)
            pltpu.sync_copy(
                m, out1.at[pl.ds(pl.multiple_of((h * NPAD + base) * C, 8), NPT * C)])

    return stage_b


_stage_b4 = _make_stage_b(4)
_stage_b1 = _make_stage_b(1)


def _mm(a3, w4):
    KH, M, Cin = a3.shape
    _, HO, _, CO = w4.shape
    BM = 640

    def body(a_ref, w_ref, o_ref):
        kh = pl.program_id(2)
        part = jnp.dot(a_ref[0], w_ref[0, 0],
                       preferred_element_type=_f32)

        @pl.when(kh == 0)
        def _():
            o_ref[0] = part

        @pl.when(kh != 0)
        def _():
            o_ref[0] = o_ref[0] + part

    return pl.pallas_call(
        body,
        grid=(HO, M // BM, KH),
        in_specs=[
            pl.BlockSpec((1, BM, Cin), lambda ho, i, kh: (kh, i, 0)),
            pl.BlockSpec((1, 1, Cin, CO), lambda ho, i, kh: (kh, ho, 0, 0)),
        ],
        out_specs=pl.BlockSpec((1, BM, CO), lambda ho, i, kh: (ho, i, 0)),
        out_shape=jax.ShapeDtypeStruct((HO, M, CO), _f32),
    )(a3, w4)


def _head(h2, llW, llb, flW, flb):
    M = h2.shape[0]
    BM = 640

    def body(h_ref, lw_ref, lb_ref, fw_ref, fb_ref, o_ref):
        t = jnp.dot(h_ref[...], lw_ref[...], preferred_element_type=_f32)
        t = t + lb_ref[...]
        u = jnp.dot(t, fw_ref[...], preferred_element_type=_f32)
        u = u + fb_ref[...]
        o_ref[...] = jax.nn.sigmoid(u)

    return pl.pallas_call(
        body,
        grid=(M // BM,),
        in_specs=[
            pl.BlockSpec((BM, 256), lambda i: (i, 0)),
            pl.BlockSpec((256, 128), lambda i: (0, 0)),
            pl.BlockSpec((1, 128), lambda i: (0, 0)),
            pl.BlockSpec((128, 1), lambda i: (0, 0)),
            pl.BlockSpec((1, 1), lambda i: (0, 0)),
        ],
        out_specs=pl.BlockSpec((BM, 1), lambda i: (i, 0)),
        out_shape=jax.ShapeDtypeStruct((M, 1), _f32),
    )(h2, llW, llb, flW, flb)


def kernel(x, edge_index, edge_attr, batch,
           Wl0, Wr0, We0, att0, b0,
           Wl1, Wr1, We1, att1, b1,
           Wl2, Wr2, We2, att2, b2,
           llW, llb, flW, flb):
    src = edge_index[0]
    dst = edge_index[1]

    cnt8 = _g1(dst)
    perm, srcp, dstp, offs8 = _g2(dst, src, cnt8)

    x_p = jnp.pad(x, ((0, NPAD - N), (0, 0)))
    h = x_p[None]
    ea_in = edge_attr[None]

    layer_params = [
        (Wl0, Wr0, We0, att0, b0, 4, _stage_b4),
        (Wl1, Wr1, We1, att1, b1, 4, _stage_b4),
        (Wl2, Wr2, We2, att2, b2, 1, _stage_b1),
    ]
    for Wl, Wr, We, att, b, H, sb in layer_params:
        KH = h.shape[0]
        wl4 = Wl.reshape(KH, C, H, C).transpose(0, 2, 1, 3)
        wr4 = Wr.reshape(KH, C, H, C).transpose(0, 2, 1, 3)
        we4 = We.reshape(1, 16, H, C).transpose(0, 2, 1, 3)
        xl3 = _mm(h, wl4)
        xr3 = _mm(h, wr4)
        ea3 = _mm(ea_in, we4)
        out1, _unused = sb(
            xl3.reshape(H * NPAD, C), xr3.reshape(H * NPAD, C),
            ea3.reshape(H * E, C), att.reshape(H * C),
            b.reshape(H * C), srcp, dstp, perm, cnt8, offs8)
        h = out1.reshape(H, NPAD, C)

    out = _head(h[0], llW, llb.reshape(1, 128), flW, flb.reshape(1, 1))
    return out[:N]
